# baseline (speedup 1.0000x reference)
import functools

import jax
import jax.numpy as jnp
import numpy as np
from jax.experimental import pallas as pl
from jax.experimental.pallas import tpu as pltpu

_THR = 0.75
_NOC = -1


def _nn_kernel(x_ref, c_ref, a_ref, o_ref,
               xs_ref, x2_ref, c2m_ref, min_ref, jwin_ref, vbest_ref,
               *, nt, bn):
    i = pl.program_id(0)
    j = pl.program_id(1)

    @pl.when(j == 0)
    def _row_init():
        x = x_ref[...]
        xs_ref[...] = x * -2.0
        x2_ref[...] = jnp.sum(x * x, axis=1, keepdims=True)
        min_ref[...] = jnp.full_like(min_ref, jnp.inf)
        jwin_ref[...] = jnp.zeros_like(jwin_ref)

    @pl.when(i == 0)
    def _code_init():
        c = c_ref[...]
        c2 = jnp.sum(c * c, axis=1)[None, :]
        c2m_ref[j] = jnp.where(a_ref[...] > 0, c2, jnp.inf)

    s = jax.lax.dot_general(xs_ref[...], c_ref[...], (((1,), (1,)), ((), ())),
                            preferred_element_type=jnp.float32)
    v = s + c2m_ref[j]
    tmin = jnp.min(v, axis=1, keepdims=True)
    better = tmin < min_ref[...]
    min_ref[...] = jnp.where(better, tmin, min_ref[...])
    jwin_ref[...] = jnp.where(better, j, jwin_ref[...])
    vbest_ref[...] = jnp.where(better, v, vbest_ref[...])

    @pl.when(j == nt - 1)
    def _fin():
        vb = vbest_ref[...]
        rm = min_ref[...]
        iota = jax.lax.broadcasted_iota(jnp.int32, vb.shape, 1)
        targ = jnp.min(jnp.where(vb == rm, iota, bn), axis=1, keepdims=True)
        idx = jwin_ref[...] * bn + targ
        mind = rm + x2_ref[...]
        o_ref[...] = jnp.where(mind <= _THR, idx, _NOC).astype(jnp.int32)


def kernel(imgs, patch_size, codes, active):
    B, C, T, H, W = imgs.shape
    N, D = codes.shape
    p = int(np.sqrt(D // C))
    Hp, Wp = H // p, W // p
    x = imgs.reshape(B, C, T, Hp, p, Wp, p).transpose(0, 2, 3, 5, 4, 6, 1)
    x = x.reshape(-1, D)
    M = x.shape[0]

    BN = 512
    BM = next((b for b in (1536, 1152, 768, 512, 256, 128, 8) if M % b == 0), M)
    MT, NT = M // BM, N // BN
    amask = active.astype(jnp.float32).reshape(1, N)

    out = pl.pallas_call(
        functools.partial(_nn_kernel, nt=NT, bn=BN),
        grid=(MT, NT),
        in_specs=[
            pl.BlockSpec((BM, D), lambda i, j: (i, 0)),
            pl.BlockSpec((BN, D), lambda i, j: (j, 0)),
            pl.BlockSpec((1, BN), lambda i, j: (0, j)),
        ],
        out_specs=pl.BlockSpec((BM, 1), lambda i, j: (i, 0)),
        out_shape=jax.ShapeDtypeStruct((M, 1), jnp.int32),
        scratch_shapes=[
            pltpu.VMEM((BM, D), jnp.float32),
            pltpu.VMEM((BM, 1), jnp.float32),
            pltpu.VMEM((NT, 1, BN), jnp.float32),
            pltpu.VMEM((BM, 1), jnp.float32),
            pltpu.VMEM((BM, 1), jnp.int32),
            pltpu.VMEM((BM, BN), jnp.float32),
        ],
    )(x, codes, amask)
    return out.reshape(B, T, Hp, Wp)


# speedup vs baseline: 1.1815x; 1.1815x over previous
import functools

import jax
import jax.numpy as jnp
import numpy as np
from jax.experimental import pallas as pl
from jax.experimental.pallas import tpu as pltpu

_THR = 0.75
_NOC = -1


def _nn_kernel(x_ref, c_ref, a_ref, o_ref,
               xs_ref, x2_ref, c2m_ref, min_ref, arg_ref,
               *, nt, bn):
    i = pl.program_id(0)
    j = pl.program_id(1)

    @pl.when(j == 0)
    def _row_init():
        x = x_ref[...]
        xs_ref[...] = x * -2.0
        x2_ref[...] = jnp.sum(x * x, axis=1, keepdims=True)
        min_ref[...] = jnp.full_like(min_ref, jnp.inf)
        arg_ref[...] = jnp.zeros_like(arg_ref)

    @pl.when(i == 0)
    def _code_init():
        c = c_ref[...]
        c2 = jnp.sum(c * c, axis=1)[None, :]
        c2m_ref[j] = jnp.where(a_ref[...] > 0, c2, jnp.inf)

    s = jax.lax.dot_general(xs_ref[...], c_ref[...], (((1,), (1,)), ((), ())),
                            preferred_element_type=jnp.float32)
    v = s + c2m_ref[j]
    tmin = jnp.min(v, axis=1, keepdims=True)
    iota = jax.lax.broadcasted_iota(jnp.int32, v.shape, 1)
    targ = jnp.min(jnp.where(v == tmin, iota, bn), axis=1, keepdims=True) + j * bn
    better = tmin < min_ref[...]
    arg_ref[...] = jnp.where(better, targ, arg_ref[...])
    min_ref[...] = jnp.where(better, tmin, min_ref[...])

    @pl.when(j == nt - 1)
    def _fin():
        mind = min_ref[...] + x2_ref[...]
        o_ref[...] = jnp.where(mind <= _THR, arg_ref[...], _NOC).astype(jnp.int32)


def kernel(imgs, patch_size, codes, active):
    B, C, T, H, W = imgs.shape
    N, D = codes.shape
    p = int(np.sqrt(D // C))
    Hp, Wp = H // p, W // p
    x = imgs.reshape(B, C, T, Hp, p, Wp, p).transpose(0, 2, 3, 5, 4, 6, 1)
    x = x.reshape(-1, D)
    M = x.shape[0]

    BN = 512
    BM = next((b for b in (1536, 1152, 768, 512, 256, 128, 8) if M % b == 0), M)
    MT, NT = M // BM, N // BN
    amask = active.astype(jnp.float32).reshape(1, N)

    out = pl.pallas_call(
        functools.partial(_nn_kernel, nt=NT, bn=BN),
        grid=(MT, NT),
        in_specs=[
            pl.BlockSpec((BM, D), lambda i, j: (i, 0)),
            pl.BlockSpec((BN, D), lambda i, j: (j, 0)),
            pl.BlockSpec((1, BN), lambda i, j: (0, j)),
        ],
        out_specs=pl.BlockSpec((BM, 1), lambda i, j: (i, 0)),
        out_shape=jax.ShapeDtypeStruct((M, 1), jnp.int32),
        scratch_shapes=[
            pltpu.VMEM((BM, D), jnp.float32),
            pltpu.VMEM((BM, 1), jnp.float32),
            pltpu.VMEM((NT, 1, BN), jnp.float32),
            pltpu.VMEM((BM, 1), jnp.float32),
            pltpu.VMEM((BM, 1), jnp.int32),
        ],
    )(x, codes, amask)
    return out.reshape(B, T, Hp, Wp)
